# Initial kernel scaffold; baseline (speedup 1.0000x reference)
import functools

import jax
import jax.numpy as jnp
from jax import lax
from jax.experimental import pallas as pl
from jax.experimental.pallas import tpu as pltpu
from jax.experimental.pallas import tpu_sc as plsc

N = 50000
E = 800000
D = 64
H = 32
B = 4096
NSUB = 20
SUB = 100
C = NSUB * SUB
EPT = E // 16
NCHUNK = EPT // C
RPT = N // 16
DPT = B // 16


def _body(urow, ucol, uval, irow, icol, ival, users2, pos2, neg2, utab, itab,
          out_u, out_p, out_n,
          cols_v, rows_v, vals_v, gath_v, idx_v, drows_v, ego_v, acc):
    c = lax.axis_index("c")
    s = lax.axis_index("s")

    zero16 = jnp.zeros((16,), jnp.float32)

    def zero_acc():
        def zrow(e, carry):
            gath_v[e, 0:16] = zero16
            gath_v[e, 16:32] = zero16
            return carry
        lax.fori_loop(0, C, zrow, 0)
        base = s * RPT
        pltpu.sync_copy(gath_v, acc.at[pl.ds(base, C)])
        pltpu.sync_copy(gath_v.at[pl.ds(0, RPT - C)],
                        acc.at[pl.ds(base + C, RPT - C)])

    def side(tab, row2, col2, val1):
        def chunk(k, carry):
            rbase = s * (EPT // SUB) + k * NSUB
            ebase = s * EPT + k * C
            pltpu.sync_copy(col2.at[pl.ds(rbase, NSUB)], cols_v)
            pltpu.sync_copy(row2.at[pl.ds(rbase, NSUB)], rows_v)
            pltpu.sync_copy(val1.at[pl.ds(ebase, C)], vals_v)

            def gsub(j, cg):
                pltpu.sync_copy(tab.at[c].at[cols_v.at[j]],
                                gath_v.at[pl.ds(j * SUB, SUB)])
                return cg
            lax.fori_loop(0, NSUB, gsub, 0)

            def scale(e, cs):
                v = vals_v[e]
                gath_v[e, 0:16] = gath_v[e, 0:16] * v
                gath_v[e, 16:32] = gath_v[e, 16:32] * v
                return cs
            lax.fori_loop(0, C, scale, 0)

            def ssub(j, cg):
                pltpu.sync_copy(gath_v.at[pl.ds(j * SUB, SUB)],
                                acc.at[rows_v.at[j]], add=True)
                return cg
            lax.fori_loop(0, NSUB, ssub, 0)
            return carry
        lax.fori_loop(0, NCHUNK, chunk, 0)

    def drain(tab, idx2, out):
        pltpu.sync_copy(idx2.at[pl.ds(2 * s, 2)], idx_v)
        for r in range(2):
            pltpu.sync_copy(acc.at[idx_v.at[r]],
                            drows_v.at[pl.ds(128 * r, 128)])
            pltpu.sync_copy(tab.at[c].at[idx_v.at[r]],
                            ego_v.at[pl.ds(128 * r, 128)])

        def comb(e, cc):
            drows_v[e, 0:16] = (ego_v[e, 0:16] * 0.25
                                + drows_v[e, 0:16] * 0.75)
            drows_v[e, 16:32] = (ego_v[e, 16:32] * 0.25
                                 + drows_v[e, 16:32] * 0.75)
            return cc
        lax.fori_loop(0, DPT, comb, 0)
        pltpu.sync_copy(drows_v, out.at[c].at[pl.ds(DPT * s, DPT)])

    zero_acc()
    plsc.subcore_barrier()
    side(utab, urow, ucol, uval)
    plsc.subcore_barrier()
    drain(utab, users2, out_u)
    plsc.subcore_barrier()
    zero_acc()
    plsc.subcore_barrier()
    side(itab, irow, icol, ival)
    plsc.subcore_barrier()
    drain(itab, pos2, out_p)
    drain(itab, neg2, out_n)


@jax.jit
def _run(urow, ucol, uval, irow, icol, ival, users2, pos2, neg2, utab, itab):
    f = pl.kernel(
        _body,
        out_type=(
            jax.ShapeDtypeStruct((2, B, H), jnp.float32),
            jax.ShapeDtypeStruct((2, B, H), jnp.float32),
            jax.ShapeDtypeStruct((2, B, H), jnp.float32),
        ),
        mesh=plsc.VectorSubcoreMesh(core_axis_name="c", subcore_axis_name="s"),
        scratch_types=[
            pltpu.VMEM((NSUB, SUB), jnp.int32),
            pltpu.VMEM((NSUB, SUB), jnp.int32),
            pltpu.VMEM((C,), jnp.float32),
            pltpu.VMEM((C, H), jnp.float32),
            pltpu.VMEM((2, 128), jnp.int32),
            pltpu.VMEM((DPT, H), jnp.float32),
            pltpu.VMEM((DPT, H), jnp.float32),
            pltpu.VMEM_SHARED((N, H), jnp.float32),
        ],
    )
    return f(urow, ucol, uval, irow, icol, ival, users2, pos2, neg2, utab, itab)


def kernel(user_adj_indices, user_adj_values, item_adj_indices,
           item_adj_values, users, pos_items, neg_items, user_emb, item_emb):
    urow = user_adj_indices[0].reshape(E // SUB, SUB)
    ucol = user_adj_indices[1].reshape(E // SUB, SUB)
    irow = item_adj_indices[0].reshape(E // SUB, SUB)
    icol = item_adj_indices[1].reshape(E // SUB, SUB)
    users2 = users.reshape(B // 128, 128)
    pos2 = pos_items.reshape(B // 128, 128)
    neg2 = neg_items.reshape(B // 128, 128)
    utab = jnp.stack([user_emb[:, :H], user_emb[:, H:]])
    itab = jnp.stack([item_emb[:, :H], item_emb[:, H:]])
    out_u, out_p, out_n = _run(urow, ucol, user_adj_values,
                               irow, icol, item_adj_values,
                               users2, pos2, neg2, utab, itab)
    u = jnp.concatenate([out_u[0], out_u[1]], axis=1)
    p = jnp.concatenate([out_p[0], out_p[1]], axis=1)
    n = jnp.concatenate([out_n[0], out_n[1]], axis=1)
    return (u, p, n)


# speedup vs baseline: 3.1902x; 3.1902x over previous
import jax
import jax.numpy as jnp
from jax import lax
from jax.experimental import pallas as pl
from jax.experimental.pallas import tpu as pltpu
from jax.experimental.pallas import tpu_sc as plsc

N = 50000
E = 800000
D = 64
H = 16
B = 4096
NSUB = 16
SUB = 128
C = NSUB * SUB
EP = 819200
EPT = EP // 16
NCHUNK = EPT // C
RPT = N // 16
DPT = B // 16


def _body(urow, ucol, uval, irow, icol, ival, users1, pos1, neg1, utab, itab,
          out_u, out_p, out_n,
          cols_v, rows_v, vals_v, gath_v, idx_v, drows_v, ego_v, acc):
    c = lax.axis_index("c")
    s = lax.axis_index("s")

    zero16 = jnp.zeros((16,), jnp.float32)

    def zero_acc():
        def zrow(e, carry):
            gath_v[e, 0:16] = zero16
            return carry
        lax.fori_loop(0, C, zrow, 0)
        base = s * RPT
        pltpu.sync_copy(gath_v, acc.at[pl.ds(base, C)])
        pltpu.sync_copy(gath_v.at[pl.ds(0, RPT - C)],
                        acc.at[pl.ds(base + C, RPT - C)])

    def side(tab, q, row2, col1, val1):
        def chunk(k, carry):
            rbase = s * (EPT // SUB) + k * NSUB
            ebase = s * EPT + k * C
            pltpu.sync_copy(col1.at[pl.ds(ebase, C)], cols_v)
            pltpu.sync_copy(row2.at[pl.ds(rbase, NSUB)], rows_v)
            pltpu.sync_copy(val1.at[pl.ds(ebase, C)], vals_v)

            def gsub(j, cg):
                pltpu.sync_copy(tab.at[q].at[cols_v.at[pl.ds(j * SUB, SUB)]],
                                gath_v.at[pl.ds(j * SUB, SUB)])
                return cg
            lax.fori_loop(0, NSUB, gsub, 0)

            def scale(g, cs):
                v16 = vals_v[pl.ds(16 * g, 16)]
                for i in range(16):
                    e = 16 * g + i
                    gath_v[e, 0:16] = gath_v[e, 0:16] * v16[i]
                return cs
            lax.fori_loop(0, C // 16, scale, 0)

            def ssub(j, cg):
                pltpu.sync_copy(gath_v.at[pl.ds(j * SUB, SUB)],
                                acc.at[rows_v.at[j]], add=True)
                return cg
            lax.fori_loop(0, NSUB, ssub, 0)
            return carry
        lax.fori_loop(0, NCHUNK, chunk, 0)

    def drain(tab, q, idx1, out):
        pltpu.sync_copy(idx1.at[pl.ds(DPT * s, DPT)], idx_v)
        for r in range(2):
            pltpu.sync_copy(acc.at[idx_v.at[pl.ds(128 * r, 128)]],
                            drows_v.at[pl.ds(128 * r, 128)])
            pltpu.sync_copy(tab.at[q].at[idx_v.at[pl.ds(128 * r, 128)]],
                            ego_v.at[pl.ds(128 * r, 128)])

        def comb(e, cc):
            drows_v[e, 0:16] = (ego_v[e, 0:16] * 0.25
                                + drows_v[e, 0:16] * 0.75)
            return cc
        lax.fori_loop(0, DPT, comb, 0)
        pltpu.sync_copy(drows_v, out.at[q].at[pl.ds(DPT * s, DPT)])

    for p in range(2):
        q = 2 * c + p
        zero_acc()
        plsc.subcore_barrier()
        side(utab, q, urow, ucol, uval)
        plsc.subcore_barrier()
        drain(utab, q, users1, out_u)
        plsc.subcore_barrier()
        zero_acc()
        plsc.subcore_barrier()
        side(itab, q, irow, icol, ival)
        plsc.subcore_barrier()
        drain(itab, q, pos1, out_p)
        drain(itab, q, neg1, out_n)
        plsc.subcore_barrier()


@jax.jit
def _run(urow, ucol, uval, irow, icol, ival, users1, pos1, neg1, utab, itab):
    f = pl.kernel(
        _body,
        out_type=(
            jax.ShapeDtypeStruct((4, B, H), jnp.float32),
            jax.ShapeDtypeStruct((4, B, H), jnp.float32),
            jax.ShapeDtypeStruct((4, B, H), jnp.float32),
        ),
        mesh=plsc.VectorSubcoreMesh(core_axis_name="c", subcore_axis_name="s"),
        compiler_params=pltpu.CompilerParams(use_tc_tiling_on_sc=False),
        scratch_types=[
            pltpu.VMEM((C,), jnp.int32),
            pltpu.VMEM((NSUB, SUB), jnp.int32),
            pltpu.VMEM((C,), jnp.float32),
            pltpu.VMEM((C, H), jnp.float32),
            pltpu.VMEM((DPT,), jnp.int32),
            pltpu.VMEM((DPT, H), jnp.float32),
            pltpu.VMEM((DPT, H), jnp.float32),
            pltpu.VMEM_SHARED((N, H), jnp.float32),
        ],
    )
    return f(urow, ucol, uval, irow, icol, ival, users1, pos1, neg1, utab, itab)


def kernel(user_adj_indices, user_adj_values, item_adj_indices,
           item_adj_values, users, pos_items, neg_items, user_emb, item_emb):
    zpad = jnp.zeros((EP - E,), jnp.int32)
    urow = jnp.concatenate([user_adj_indices[0], zpad]).reshape(EP // SUB, SUB)
    ucol = jnp.concatenate([user_adj_indices[1], zpad])
    irow = jnp.concatenate([item_adj_indices[0], zpad]).reshape(EP // SUB, SUB)
    icol = jnp.concatenate([item_adj_indices[1], zpad])
    vpad = jnp.zeros((EP - E,), jnp.float32)
    uvalp = jnp.concatenate([user_adj_values, vpad])
    ivalp = jnp.concatenate([item_adj_values, vpad])
    utab = jnp.stack([user_emb[:, 16 * i:16 * i + 16] for i in range(4)])
    itab = jnp.stack([item_emb[:, 16 * i:16 * i + 16] for i in range(4)])
    out_u, out_p, out_n = _run(urow, ucol, uvalp,
                               irow, icol, ivalp,
                               users, pos_items, neg_items, utab, itab)
    u = jnp.concatenate([out_u[0], out_u[1], out_u[2], out_u[3]], axis=1)
    p = jnp.concatenate([out_p[0], out_p[1], out_p[2], out_p[3]], axis=1)
    n = jnp.concatenate([out_n[0], out_n[1], out_n[2], out_n[3]], axis=1)
    return (u, p, n)


# speedup vs baseline: 5.9449x; 1.8635x over previous
import jax
import jax.numpy as jnp
from jax import lax
from jax.experimental import pallas as pl
from jax.experimental.pallas import tpu as pltpu
from jax.experimental.pallas import tpu_sc as plsc

N = 50000
E = 800000
D = 64
H = 16
B = 4096
NSUB = 8
SUB = 128
C = NSUB * SUB
EP = 819200
EPT = EP // 16
NCHUNK = EPT // C
NCHT = EP // C
RPT = N // 16
DPT = B // 16
PKR = 3 * NSUB


def _body(pk_u, pk_i, users1, pos1, neg1, utab, itab,
          out_u, out_p, out_n,
          ibufA, ibufB, gbufA, gbufB, rowsA, rowsB, idx_v, acc,
          sem_g, sem_s):
    c = lax.axis_index("c")
    s = lax.axis_index("s")

    zero16 = jnp.zeros((16,), jnp.float32)

    def zero_acc():
        def zrow(e, carry):
            gbufA[e, 0:16] = zero16
            return carry
        lax.fori_loop(0, C, zrow, 0)

        def zcp(i, carry):
            pltpu.sync_copy(gbufA, acc.at[pl.ds(s * RPT + i * C, C)])
            return carry
        lax.fori_loop(0, RPT // C, zcp, 0)
        pltpu.sync_copy(gbufA.at[pl.ds(0, RPT - (RPT // C) * C)],
                        acc.at[pl.ds(s * RPT + (RPT // C) * C,
                                     RPT - (RPT // C) * C)])

    def side(tab, q, pk):
        kk0 = s * NCHUNK

        def fire_gathers(ibuf, gbuf):
            def g1(j, cg):
                pltpu.async_copy(
                    tab.at[q].at[ibuf.at[NSUB + j]],
                    gbuf.at[pl.ds(j * SUB, SUB)],
                    sem_g)
                return cg
            lax.fori_loop(0, NSUB, g1, 0)

        def drain_gathers(gbuf):
            pltpu.make_async_copy(tab.at[q].at[pl.ds(0, C)],
                                  gbuf, sem_g).wait()

        def scale_and_scatter(ibuf, gbuf, rows_v):
            def cpr(i, cg):
                r = i // 8
                o = 16 * (i - 8 * r)
                rows_v[r, pl.ds(o, 16)] = ibuf[r, pl.ds(o, 16)]
                return cg
            lax.fori_loop(0, NSUB * 8, cpr, 0)

            def scale(gidx, cs):
                jr = gidx // 8
                g2 = gidx - 8 * jr
                v16 = plsc.bitcast(
                    ibuf[2 * NSUB + jr, pl.ds(16 * g2, 16)], jnp.float32)
                for i in range(16):
                    e = jr * SUB + g2 * 16 + i
                    gbuf[e, 0:16] = gbuf[e, 0:16] * v16[i]
                return cs
            lax.fori_loop(0, C // 16, scale, 0)

            def s1(j, cg):
                pltpu.async_copy(
                    gbuf.at[pl.ds(j * SUB, SUB)],
                    acc.at[rows_v.at[j]],
                    sem_s, add=True)
                return cg
            lax.fori_loop(0, NSUB, s1, 0)

        def drain_scatters(gbuf):
            pltpu.make_async_copy(gbuf, acc.at[pl.ds(0, C)], sem_s).wait()

        pltpu.sync_copy(pk.at[kk0], ibufA)
        fire_gathers(ibufA, gbufA)

        def chunk(k, carry):
            par = lax.rem(k, 2)

            def step(ibuf, gbuf, rows_v, ibufN, gbufN):
                drain_gathers(gbuf)

                @pl.when(k >= 1)
                def _():
                    drain_scatters(gbufN)

                @pl.when(k < NCHUNK - 1)
                def _():
                    pltpu.sync_copy(pk.at[kk0 + k + 1], ibufN)
                    fire_gathers(ibufN, gbufN)

                scale_and_scatter(ibuf, gbuf, rows_v)

            @pl.when(par == 0)
            def _():
                step(ibufA, gbufA, rowsA, ibufB, gbufB)

            @pl.when(par == 1)
            def _():
                step(ibufB, gbufB, rowsB, ibufA, gbufA)

            return carry
        lax.fori_loop(0, NCHUNK, chunk, 0)

        if (NCHUNK - 1) % 2 == 0:
            drain_scatters(gbufA)
        else:
            drain_scatters(gbufB)

    def drain(tab, q, idx1, out):
        pltpu.sync_copy(idx1.at[pl.ds(DPT * s, DPT)], idx_v)
        for r in range(2):
            pltpu.sync_copy(acc.at[idx_v.at[pl.ds(128 * r, 128)]],
                            gbufA.at[pl.ds(128 * r, 128)])
            pltpu.sync_copy(tab.at[q].at[idx_v.at[pl.ds(128 * r, 128)]],
                            gbufB.at[pl.ds(128 * r, 128)])

        def comb(e, cc):
            gbufA[e, 0:16] = (gbufB[e, 0:16] * 0.25
                              + gbufA[e, 0:16] * 0.75)
            return cc
        lax.fori_loop(0, DPT, comb, 0)
        pltpu.sync_copy(gbufA.at[pl.ds(0, DPT)],
                        out.at[q].at[pl.ds(DPT * s, DPT)])

    for p in range(2):
        q = 2 * c + p
        zero_acc()
        plsc.subcore_barrier()
        side(utab, q, pk_u)
        plsc.subcore_barrier()
        drain(utab, q, users1, out_u)
        plsc.subcore_barrier()
        zero_acc()
        plsc.subcore_barrier()
        side(itab, q, pk_i)
        plsc.subcore_barrier()
        drain(itab, q, pos1, out_p)
        drain(itab, q, neg1, out_n)
        plsc.subcore_barrier()


@jax.jit
def _run(pk_u, pk_i, users1, pos1, neg1, utab, itab):
    f = pl.kernel(
        _body,
        out_type=(
            jax.ShapeDtypeStruct((4, B, H), jnp.float32),
            jax.ShapeDtypeStruct((4, B, H), jnp.float32),
            jax.ShapeDtypeStruct((4, B, H), jnp.float32),
        ),
        mesh=plsc.VectorSubcoreMesh(core_axis_name="c", subcore_axis_name="s"),
        compiler_params=pltpu.CompilerParams(use_tc_tiling_on_sc=False,
                                             needs_layout_passes=False),
        scratch_types=[
            pltpu.VMEM((PKR, SUB), jnp.int32),
            pltpu.VMEM((PKR, SUB), jnp.int32),
            pltpu.VMEM((C, H), jnp.float32),
            pltpu.VMEM((C, H), jnp.float32),
            pltpu.VMEM((NSUB, SUB), jnp.int32),
            pltpu.VMEM((NSUB, SUB), jnp.int32),
            pltpu.VMEM((DPT,), jnp.int32),
            pltpu.VMEM_SHARED((N, H), jnp.float32),
            pltpu.SemaphoreType.DMA,
            pltpu.SemaphoreType.DMA,
        ],
    )
    return f(pk_u, pk_i, users1, pos1, neg1, utab, itab)


def _pack(indices, values):
    zpad = jnp.zeros((EP - E,), jnp.int32)
    rows = jnp.concatenate([indices[0], zpad]).reshape(NCHT, NSUB, SUB)
    cols = jnp.concatenate([indices[1], zpad]).reshape(NCHT, NSUB, SUB)
    vals = jnp.concatenate(
        [lax.bitcast_convert_type(values, jnp.int32), zpad]
    ).reshape(NCHT, NSUB, SUB)
    return jnp.concatenate([rows, cols, vals], axis=1)


def kernel(user_adj_indices, user_adj_values, item_adj_indices,
           item_adj_values, users, pos_items, neg_items, user_emb, item_emb):
    pk_u = _pack(user_adj_indices, user_adj_values)
    pk_i = _pack(item_adj_indices, item_adj_values)
    utab = jnp.stack([user_emb[:, 16 * i:16 * i + 16] for i in range(4)])
    itab = jnp.stack([item_emb[:, 16 * i:16 * i + 16] for i in range(4)])
    out_u, out_p, out_n = _run(pk_u, pk_i, users, pos_items, neg_items,
                               utab, itab)
    u = jnp.concatenate([out_u[0], out_u[1], out_u[2], out_u[3]], axis=1)
    p = jnp.concatenate([out_p[0], out_p[1], out_p[2], out_p[3]], axis=1)
    n = jnp.concatenate([out_n[0], out_n[1], out_n[2], out_n[3]], axis=1)
    return (u, p, n)


# speedup vs baseline: 5.9620x; 1.0029x over previous
import jax
import jax.numpy as jnp
from jax import lax
from jax.experimental import pallas as pl
from jax.experimental.pallas import tpu as pltpu
from jax.experimental.pallas import tpu_sc as plsc

N = 50000
E = 800000
D = 64
H = 16
B = 4096
NSUB = 8
SUB = 128
C = NSUB * SUB
EP = 819200
EPT = EP // 16
NCHUNK = EPT // C
NCHT = EP // C
RPT = N // 16
DPT = B // 16
PKR = 3 * NSUB


def _body(pk_u, pk_i, users1, pos1, neg1, utab, itab,
          out_u, out_p, out_n,
          ibufA, ibufB, gbufA, gbufB, idx_v, acc,
          sem_g, sem_s):
    c = lax.axis_index("c")
    s = lax.axis_index("s")

    zero16 = jnp.zeros((16,), jnp.float32)

    def zero_acc():
        def zrow(e, carry):
            gbufA[e, 0:16] = zero16
            return carry
        lax.fori_loop(0, C, zrow, 0)

        def zcp(i, carry):
            pltpu.sync_copy(gbufA, acc.at[pl.ds(s * RPT + i * C, C)])
            return carry
        lax.fori_loop(0, RPT // C, zcp, 0)
        pltpu.sync_copy(gbufA.at[pl.ds(0, RPT - (RPT // C) * C)],
                        acc.at[pl.ds(s * RPT + (RPT // C) * C,
                                     RPT - (RPT // C) * C)])

    def side(tab, q, pk):
        kk0 = s * NCHUNK

        def fire_gathers(ibuf, gbuf):
            def g1(j, cg):
                pltpu.async_copy(
                    tab.at[q].at[ibuf.at[NSUB + j]],
                    gbuf.at[pl.ds(j * SUB, SUB)],
                    sem_g)
                return cg
            lax.fori_loop(0, NSUB, g1, 0)

        def drain_gathers(gbuf):
            pltpu.make_async_copy(tab.at[q].at[pl.ds(0, C)],
                                  gbuf, sem_g).wait()

        def scale_and_scatter(ibuf, gbuf):
            def scale(gidx, cs):
                jr = gidx // 8
                g2 = gidx - 8 * jr
                v16 = plsc.bitcast(
                    ibuf[2 * NSUB + jr, pl.ds(16 * g2, 16)], jnp.float32)
                for i in range(16):
                    e = jr * SUB + g2 * 16 + i
                    gbuf[e, 0:16] = gbuf[e, 0:16] * v16[i]
                return cs
            lax.fori_loop(0, C // 16, scale, 0, unroll=2)

            def s1(j, cg):
                pltpu.async_copy(
                    gbuf.at[pl.ds(j * SUB, SUB)],
                    acc.at[ibuf.at[j]],
                    sem_s, add=True)
                return cg
            lax.fori_loop(0, NSUB, s1, 0)

        def drain_scatters(gbuf):
            pltpu.make_async_copy(gbuf, acc.at[pl.ds(0, C)], sem_s).wait()

        pltpu.sync_copy(pk.at[kk0], ibufA)
        fire_gathers(ibufA, gbufA)

        def chunk(k, carry):
            par = lax.rem(k, 2)

            def step(ibuf, gbuf, ibufN, gbufN):
                drain_gathers(gbuf)

                @pl.when(k >= 1)
                def _():
                    drain_scatters(gbufN)

                @pl.when(k < NCHUNK - 1)
                def _():
                    pltpu.sync_copy(pk.at[kk0 + k + 1], ibufN)
                    fire_gathers(ibufN, gbufN)

                scale_and_scatter(ibuf, gbuf)

            @pl.when(par == 0)
            def _():
                step(ibufA, gbufA, ibufB, gbufB)

            @pl.when(par == 1)
            def _():
                step(ibufB, gbufB, ibufA, gbufA)

            return carry
        lax.fori_loop(0, NCHUNK, chunk, 0)

        if (NCHUNK - 1) % 2 == 0:
            drain_scatters(gbufA)
        else:
            drain_scatters(gbufB)

    def drain(tab, q, idx1, out):
        pltpu.sync_copy(idx1.at[pl.ds(DPT * s, DPT)], idx_v)
        for r in range(2):
            pltpu.sync_copy(acc.at[idx_v.at[pl.ds(128 * r, 128)]],
                            gbufA.at[pl.ds(128 * r, 128)])
            pltpu.sync_copy(tab.at[q].at[idx_v.at[pl.ds(128 * r, 128)]],
                            gbufB.at[pl.ds(128 * r, 128)])

        def comb(e, cc):
            gbufA[e, 0:16] = (gbufB[e, 0:16] * 0.25
                              + gbufA[e, 0:16] * 0.75)
            return cc
        lax.fori_loop(0, DPT, comb, 0)
        pltpu.sync_copy(gbufA.at[pl.ds(0, DPT)],
                        out.at[q].at[pl.ds(DPT * s, DPT)])

    for p in range(2):
        q = 2 * c + p
        zero_acc()
        plsc.subcore_barrier()
        side(utab, q, pk_u)
        plsc.subcore_barrier()
        drain(utab, q, users1, out_u)
        plsc.subcore_barrier()
        zero_acc()
        plsc.subcore_barrier()
        side(itab, q, pk_i)
        plsc.subcore_barrier()
        drain(itab, q, pos1, out_p)
        drain(itab, q, neg1, out_n)
        plsc.subcore_barrier()


@jax.jit
def _run(pk_u, pk_i, users1, pos1, neg1, utab, itab):
    f = pl.kernel(
        _body,
        out_type=(
            jax.ShapeDtypeStruct((4, B, H), jnp.float32),
            jax.ShapeDtypeStruct((4, B, H), jnp.float32),
            jax.ShapeDtypeStruct((4, B, H), jnp.float32),
        ),
        mesh=plsc.VectorSubcoreMesh(core_axis_name="c", subcore_axis_name="s"),
        compiler_params=pltpu.CompilerParams(use_tc_tiling_on_sc=False,
                                             needs_layout_passes=False),
        scratch_types=[
            pltpu.VMEM((PKR, SUB), jnp.int32),
            pltpu.VMEM((PKR, SUB), jnp.int32),
            pltpu.VMEM((C, H), jnp.float32),
            pltpu.VMEM((C, H), jnp.float32),
            pltpu.VMEM((DPT,), jnp.int32),
            pltpu.VMEM_SHARED((N, H), jnp.float32),
            pltpu.SemaphoreType.DMA,
            pltpu.SemaphoreType.DMA,
        ],
    )
    return f(pk_u, pk_i, users1, pos1, neg1, utab, itab)


def _pack(indices, values):
    zpad = jnp.zeros((EP - E,), jnp.int32)
    rows = jnp.concatenate([indices[0], zpad]).reshape(NCHT, NSUB, SUB)
    cols = jnp.concatenate([indices[1], zpad]).reshape(NCHT, NSUB, SUB)
    vals = jnp.concatenate(
        [lax.bitcast_convert_type(values, jnp.int32), zpad]
    ).reshape(NCHT, NSUB, SUB)
    return jnp.concatenate([rows, cols, vals], axis=1)


def kernel(user_adj_indices, user_adj_values, item_adj_indices,
           item_adj_values, users, pos_items, neg_items, user_emb, item_emb):
    pk_u = _pack(user_adj_indices, user_adj_values)
    pk_i = _pack(item_adj_indices, item_adj_values)
    utab = jnp.stack([user_emb[:, 16 * i:16 * i + 16] for i in range(4)])
    itab = jnp.stack([item_emb[:, 16 * i:16 * i + 16] for i in range(4)])
    out_u, out_p, out_n = _run(pk_u, pk_i, users, pos_items, neg_items,
                               utab, itab)
    u = jnp.concatenate([out_u[0], out_u[1], out_u[2], out_u[3]], axis=1)
    p = jnp.concatenate([out_p[0], out_p[1], out_p[2], out_p[3]], axis=1)
    n = jnp.concatenate([out_n[0], out_n[1], out_n[2], out_n[3]], axis=1)
    return (u, p, n)


# speedup vs baseline: 6.5572x; 1.0998x over previous
import jax
import jax.numpy as jnp
from jax import lax
from jax.experimental import pallas as pl
from jax.experimental.pallas import tpu as pltpu
from jax.experimental.pallas import tpu_sc as plsc

N = 50000
E = 800000
D = 64
H = 16
B = 4096
NSUB = 8
SUB = 128
C = NSUB * SUB
EP = 819200
EPT = EP // 16
NCHUNK = EPT // C
NCHT = EP // C
RPT = N // 16
DPT = B // 16
PKR = 3 * NSUB


def _body(pk_u, pk_i, users1, pos1, neg1, utab, itab,
          out_u, out_p, out_n,
          ibufA, ibufB, ibufC, gbufA, gbufB, idx_v, acc,
          sem_g, sem_s, sem_i):
    c = lax.axis_index("c")
    s = lax.axis_index("s")

    zero16 = jnp.zeros((16,), jnp.float32)

    def zero_acc():
        def zrow(e, carry):
            gbufA[e, 0:16] = zero16
            return carry
        lax.fori_loop(0, C, zrow, 0)

        def zcp(i, carry):
            pltpu.sync_copy(gbufA, acc.at[pl.ds(s * RPT + i * C, C)])
            return carry
        lax.fori_loop(0, RPT // C, zcp, 0)
        pltpu.sync_copy(gbufA.at[pl.ds(0, RPT - (RPT // C) * C)],
                        acc.at[pl.ds(s * RPT + (RPT // C) * C,
                                     RPT - (RPT // C) * C)])

    def side(tab, q, pk):
        kk0 = s * NCHUNK

        def fire_gathers(ibuf, gbuf):
            def g1(j, cg):
                pltpu.async_copy(
                    tab.at[q].at[ibuf.at[NSUB + j]],
                    gbuf.at[pl.ds(j * SUB, SUB)],
                    sem_g)
                return cg
            lax.fori_loop(0, NSUB, g1, 0)

        def drain_gathers(gbuf):
            pltpu.make_async_copy(tab.at[q].at[pl.ds(0, C)],
                                  gbuf, sem_g).wait()

        def scale_and_scatter(ibuf, gbuf):
            def scale(gidx, cs):
                jr = gidx // 8
                g2 = gidx - 8 * jr
                v16 = plsc.bitcast(
                    ibuf[2 * NSUB + jr, pl.ds(16 * g2, 16)], jnp.float32)
                for i in range(16):
                    e = jr * SUB + g2 * 16 + i
                    gbuf[e, 0:16] = gbuf[e, 0:16] * v16[i]
                return cs
            lax.fori_loop(0, C // 16, scale, 0, unroll=2)

            def s1(j, cg):
                pltpu.async_copy(
                    gbuf.at[pl.ds(j * SUB, SUB)],
                    acc.at[ibuf.at[j]],
                    sem_s, add=True)
                return cg
            lax.fori_loop(0, NSUB, s1, 0)

        def drain_scatters(gbuf):
            pltpu.make_async_copy(gbuf, acc.at[pl.ds(0, C)], sem_s).wait()

        def drain_idx(ibuf):
            pltpu.make_async_copy(pk.at[0], ibuf, sem_i).wait()

        pltpu.sync_copy(pk.at[kk0], ibufA)
        pltpu.async_copy(pk.at[kk0 + 1], ibufB, sem_i)
        fire_gathers(ibufA, gbufA)

        ibufs = (ibufA, ibufB, ibufC)
        gbufs = (gbufA, gbufB)

        def chunk(k, carry):
            par6 = lax.rem(k, 6)

            def step(ibuf, gbuf, ibufN, gbufN, ibufN2):
                drain_gathers(gbuf)

                @pl.when(k >= 1)
                def _():
                    drain_scatters(gbufN)

                @pl.when(k < NCHUNK - 1)
                def _():
                    drain_idx(ibufN)
                    fire_gathers(ibufN, gbufN)

                @pl.when(k < NCHUNK - 2)
                def _():
                    pltpu.async_copy(pk.at[kk0 + k + 2], ibufN2, sem_i)

                scale_and_scatter(ibuf, gbuf)

            for m in range(6):
                @pl.when(par6 == m)
                def _(m=m):
                    step(ibufs[m % 3], gbufs[m % 2],
                         ibufs[(m + 1) % 3], gbufs[(m + 1) % 2],
                         ibufs[(m + 2) % 3])

            return carry
        lax.fori_loop(0, NCHUNK, chunk, 0)

        if (NCHUNK - 1) % 2 == 0:
            drain_scatters(gbufA)
        else:
            drain_scatters(gbufB)

    def drain(tab, q, idx1, out):
        pltpu.sync_copy(idx1.at[pl.ds(DPT * s, DPT)], idx_v)
        for r in range(2):
            pltpu.sync_copy(acc.at[idx_v.at[pl.ds(128 * r, 128)]],
                            gbufA.at[pl.ds(128 * r, 128)])
            pltpu.sync_copy(tab.at[q].at[idx_v.at[pl.ds(128 * r, 128)]],
                            gbufB.at[pl.ds(128 * r, 128)])

        def comb(e, cc):
            gbufA[e, 0:16] = (gbufB[e, 0:16] * 0.25
                              + gbufA[e, 0:16] * 0.75)
            return cc
        lax.fori_loop(0, DPT, comb, 0)
        pltpu.sync_copy(gbufA.at[pl.ds(0, DPT)],
                        out.at[q].at[pl.ds(DPT * s, DPT)])

    for p in range(2):
        q = 2 * c + p
        zero_acc()
        plsc.subcore_barrier()
        side(utab, q, pk_u)
        plsc.subcore_barrier()
        drain(utab, q, users1, out_u)
        plsc.subcore_barrier()
        zero_acc()
        plsc.subcore_barrier()
        side(itab, q, pk_i)
        plsc.subcore_barrier()
        drain(itab, q, pos1, out_p)
        drain(itab, q, neg1, out_n)
        plsc.subcore_barrier()


@jax.jit
def _run(pk_u, pk_i, users1, pos1, neg1, utab, itab):
    f = pl.kernel(
        _body,
        out_type=(
            jax.ShapeDtypeStruct((4, B, H), jnp.float32),
            jax.ShapeDtypeStruct((4, B, H), jnp.float32),
            jax.ShapeDtypeStruct((4, B, H), jnp.float32),
        ),
        mesh=plsc.VectorSubcoreMesh(core_axis_name="c", subcore_axis_name="s"),
        compiler_params=pltpu.CompilerParams(use_tc_tiling_on_sc=False,
                                             needs_layout_passes=False),
        scratch_types=[
            pltpu.VMEM((PKR, SUB), jnp.int32),
            pltpu.VMEM((PKR, SUB), jnp.int32),
            pltpu.VMEM((PKR, SUB), jnp.int32),
            pltpu.VMEM((C, H), jnp.float32),
            pltpu.VMEM((C, H), jnp.float32),
            pltpu.VMEM((DPT,), jnp.int32),
            pltpu.VMEM_SHARED((N, H), jnp.float32),
            pltpu.SemaphoreType.DMA,
            pltpu.SemaphoreType.DMA,
            pltpu.SemaphoreType.DMA,
        ],
    )
    return f(pk_u, pk_i, users1, pos1, neg1, utab, itab)


def _pack(indices, values):
    zpad = jnp.zeros((EP - E,), jnp.int32)
    rows = jnp.concatenate([indices[0], zpad]).reshape(NCHT, NSUB, SUB)
    cols = jnp.concatenate([indices[1], zpad]).reshape(NCHT, NSUB, SUB)
    vals = jnp.concatenate(
        [lax.bitcast_convert_type(values, jnp.int32), zpad]
    ).reshape(NCHT, NSUB, SUB)
    return jnp.concatenate([rows, cols, vals], axis=1)


def kernel(user_adj_indices, user_adj_values, item_adj_indices,
           item_adj_values, users, pos_items, neg_items, user_emb, item_emb):
    pk_u = _pack(user_adj_indices, user_adj_values)
    pk_i = _pack(item_adj_indices, item_adj_values)
    utab = jnp.stack([user_emb[:, 16 * i:16 * i + 16] for i in range(4)])
    itab = jnp.stack([item_emb[:, 16 * i:16 * i + 16] for i in range(4)])
    out_u, out_p, out_n = _run(pk_u, pk_i, users, pos_items, neg_items,
                               utab, itab)
    u = jnp.concatenate([out_u[0], out_u[1], out_u[2], out_u[3]], axis=1)
    p = jnp.concatenate([out_p[0], out_p[1], out_p[2], out_p[3]], axis=1)
    n = jnp.concatenate([out_n[0], out_n[1], out_n[2], out_n[3]], axis=1)
    return (u, p, n)
